# Initial kernel scaffold; baseline (speedup 1.0000x reference)
import jax
import jax.numpy as jnp
from jax.experimental import pallas as pl


def kernel(z, codebook):
    raise NotImplementedError("write your pallas kernel here")


# speedup vs baseline: 1.0688x; 1.0688x over previous
import jax
import jax.numpy as jnp
from jax.experimental import pallas as pl

_NUM_CODE = 1024
_CODE_DIM = 256
_BETA = 0.1
_ROWS_PER_BLOCK = 1024


def _vq_body(z_ref, sz_ref, cb_ref, se_ref,
             oh_ref, zq_ref, idx_ref, cnt_ref, sq_ref):
    i = pl.program_id(0)
    z = z_ref[...]
    cb = cb_ref[...]
    mm = jax.lax.dot_general(z, cb, (((1,), (1,)), ((), ())),
                             preferred_element_type=jnp.float32)
    d = (sz_ref[...] + se_ref[...]) - 2.0 * mm
    m = jnp.min(d, axis=1, keepdims=True)
    lane = jax.lax.broadcasted_iota(jnp.int32, d.shape, 1)
    idx = jnp.min(jnp.where(d == m, lane, _NUM_CODE), axis=1, keepdims=True)
    oh = (lane == idx).astype(jnp.float32)
    oh_ref[...] = oh
    idx_ref[...] = idx
    zq = jax.lax.dot_general(oh, cb, (((1,), (0,)), ((), ())),
                             preferred_element_type=jnp.float32,
                             precision=jax.lax.Precision.HIGHEST)
    zq_ref[...] = z + (zq - z)
    cnt = jnp.sum(oh, axis=0, keepdims=True)
    sq = jnp.sum((zq - z) ** 2).reshape(1, 1)

    @pl.when(i == 0)
    def _():
        cnt_ref[...] = cnt
        sq_ref[...] = sq

    @pl.when(i != 0)
    def _():
        cnt_ref[...] += cnt
        sq_ref[...] += sq


def kernel(z, codebook):
    B, T, D = z.shape
    n_rows = B * T
    z_flat = z.reshape(-1, D)
    s_z = jnp.sum(z_flat ** 2, axis=1, keepdims=True)
    s_e = jnp.sum(codebook ** 2, axis=1)[None, :]

    R = _ROWS_PER_BLOCK
    grid = (n_rows // R,)
    oh, zq_st, idx, cnt, sq = pl.pallas_call(
        _vq_body,
        grid=grid,
        in_specs=[
            pl.BlockSpec((R, D), lambda i: (i, 0)),
            pl.BlockSpec((R, 1), lambda i: (i, 0)),
            pl.BlockSpec((_NUM_CODE, D), lambda i: (0, 0)),
            pl.BlockSpec((1, _NUM_CODE), lambda i: (0, 0)),
        ],
        out_specs=[
            pl.BlockSpec((R, _NUM_CODE), lambda i: (i, 0)),
            pl.BlockSpec((R, D), lambda i: (i, 0)),
            pl.BlockSpec((R, 1), lambda i: (i, 0)),
            pl.BlockSpec((1, _NUM_CODE), lambda i: (0, 0)),
            pl.BlockSpec((1, 1), lambda i: (0, 0)),
        ],
        out_shape=[
            jax.ShapeDtypeStruct((n_rows, _NUM_CODE), jnp.float32),
            jax.ShapeDtypeStruct((n_rows, D), jnp.float32),
            jax.ShapeDtypeStruct((n_rows, 1), jnp.int32),
            jax.ShapeDtypeStruct((1, _NUM_CODE), jnp.float32),
            jax.ShapeDtypeStruct((1, 1), jnp.float32),
        ],
    )(z_flat, s_z, codebook, s_e)

    v = sq[0, 0] / jnp.float32(n_rows * D)
    loss = v + _BETA * v
    e_mean = cnt[0] / jnp.float32(n_rows)
    perplexity = jnp.exp(-jnp.sum(e_mean * jnp.log(e_mean + 1e-10)))
    return (loss, zq_st.reshape(B, T, D), perplexity, oh, idx)


# speedup vs baseline: 1.2422x; 1.1622x over previous
import functools

import jax
import jax.numpy as jnp
from jax import lax
from jax.experimental import pallas as pl
from jax.experimental.pallas import tpu as pltpu
from jax.experimental.pallas import tpu_sc as plsc

_NUM_CODE = 1024
_CODE_DIM = 256
_BETA = 0.1
_ROWS_PER_BLOCK = 1024

_SC_INFO = plsc.get_sparse_core_info()
_NC = _SC_INFO.num_cores
_NS = _SC_INFO.num_subcores
_NW = _NC * _NS


def _vq_body(z_ref, sz_ref, cb_ref, se_ref,
             oh_ref, idx_ref, cnt_ref, sq_ref):
    i = pl.program_id(0)
    z = z_ref[...]
    cb = cb_ref[...]
    mm = jax.lax.dot_general(z, cb, (((1,), (1,)), ((), ())),
                             preferred_element_type=jnp.float32)
    d = (sz_ref[...] + se_ref[...]) - 2.0 * mm
    m = jnp.min(d, axis=1, keepdims=True)
    lane = jax.lax.broadcasted_iota(jnp.int32, d.shape, 1)
    idx = jnp.min(jnp.where(d == m, lane, _NUM_CODE), axis=1, keepdims=True)
    oh = (lane == idx).astype(jnp.float32)
    oh_ref[...] = oh
    idx_ref[...] = idx
    cnt = jnp.sum(oh, axis=0, keepdims=True)
    sq = jnp.sum(m).reshape(1, 1)

    @pl.when(i == 0)
    def _():
        cnt_ref[...] = cnt
        sq_ref[...] = sq

    @pl.when(i != 0)
    def _():
        cnt_ref[...] += cnt
        sq_ref[...] += sq


def _gather_rows(n_rows, chunk):
    n_chunks = n_rows // (_NW * chunk)
    mesh = plsc.VectorSubcoreMesh(core_axis_name="c", subcore_axis_name="s")

    @functools.partial(
        pl.kernel, mesh=mesh,
        out_type=jax.ShapeDtypeStruct((n_rows, _CODE_DIM), jnp.float32),
        scratch_types=[
            pltpu.VMEM((chunk,), jnp.int32),
            pltpu.VMEM((chunk, _CODE_DIM), jnp.float32),
            pltpu.SemaphoreType.DMA,
        ],
    )
    def gather(cb_hbm, idx_hbm, out_hbm, idx_v, rows_v, sem):
        wid = lax.axis_index("s") * _NC + lax.axis_index("c")
        for j in range(n_chunks):
            base = wid * (n_chunks * chunk) + j * chunk
            pltpu.sync_copy(idx_hbm.at[pl.ds(base, chunk)], idx_v)
            pltpu.async_copy(cb_hbm.at[idx_v], rows_v, sem).wait()
            pltpu.sync_copy(rows_v, out_hbm.at[pl.ds(base, chunk)])

    return gather


def kernel(z, codebook):
    B, T, D = z.shape
    n_rows = B * T
    z_flat = z.reshape(-1, D)
    s_z = jnp.sum(z_flat ** 2, axis=1, keepdims=True)
    s_e = jnp.sum(codebook ** 2, axis=1)[None, :]

    R = _ROWS_PER_BLOCK
    grid = (n_rows // R,)
    oh, idx, cnt, sq = pl.pallas_call(
        _vq_body,
        grid=grid,
        in_specs=[
            pl.BlockSpec((R, D), lambda i: (i, 0)),
            pl.BlockSpec((R, 1), lambda i: (i, 0)),
            pl.BlockSpec((_NUM_CODE, D), lambda i: (0, 0)),
            pl.BlockSpec((1, _NUM_CODE), lambda i: (0, 0)),
        ],
        out_specs=[
            pl.BlockSpec((R, _NUM_CODE), lambda i: (i, 0)),
            pl.BlockSpec((R, 1), lambda i: (i, 0)),
            pl.BlockSpec((1, _NUM_CODE), lambda i: (0, 0)),
            pl.BlockSpec((1, 1), lambda i: (0, 0)),
        ],
        out_shape=[
            jax.ShapeDtypeStruct((n_rows, _NUM_CODE), jnp.float32),
            jax.ShapeDtypeStruct((n_rows, 1), jnp.int32),
            jax.ShapeDtypeStruct((1, _NUM_CODE), jnp.float32),
            jax.ShapeDtypeStruct((1, 1), jnp.float32),
        ],
    )(z_flat, s_z, codebook, s_e)

    zq_st = _gather_rows(n_rows, 128)(codebook, idx.reshape(-1))

    v = sq[0, 0] / jnp.float32(n_rows * D)
    loss = v + _BETA * v
    e_mean = cnt[0] / jnp.float32(n_rows)
    perplexity = jnp.exp(-jnp.sum(e_mean * jnp.log(e_mean + 1e-10)))
    return (loss, zq_st.reshape(B, T, D), perplexity, oh, idx)


# speedup vs baseline: 1.2786x; 1.0293x over previous
import functools

import jax
import jax.numpy as jnp
from jax import lax
from jax.experimental import pallas as pl
from jax.experimental.pallas import tpu as pltpu
from jax.experimental.pallas import tpu_sc as plsc

_NUM_CODE = 1024
_CODE_DIM = 256
_BETA = 0.1
_ROWS_PER_BLOCK = 1024

_SC_INFO = plsc.get_sparse_core_info()
_NC = _SC_INFO.num_cores
_NS = _SC_INFO.num_subcores
_NW = _NC * _NS


def _vq_body(z_ref, sz_ref, cb_ref, se_ref,
             oh_ref, idx_ref, cnt_ref, sq_ref):
    i = pl.program_id(0)
    z = z_ref[...]
    cb = cb_ref[...]
    mm = jax.lax.dot_general(z, cb, (((1,), (1,)), ((), ())),
                             preferred_element_type=jnp.float32)
    sz_col = sz_ref[0].reshape(1, -1).swapaxes(0, 1)
    d = (sz_col + se_ref[...]) - 2.0 * mm
    m = jnp.min(d, axis=1, keepdims=True)
    lane = jax.lax.broadcasted_iota(jnp.int32, d.shape, 1)
    idx = jnp.min(jnp.where(d == m, lane, _NUM_CODE), axis=1, keepdims=True)
    oh = (lane == idx).astype(jnp.float32)
    oh_ref[...] = oh
    idx_ref[...] = idx.reshape(1, 1, -1)
    cnt = jnp.sum(oh, axis=0, keepdims=True)
    sq = jnp.sum(m).reshape(1, 1)

    @pl.when(i == 0)
    def _():
        cnt_ref[...] = cnt
        sq_ref[...] = sq

    @pl.when(i != 0)
    def _():
        cnt_ref[...] += cnt
        sq_ref[...] += sq


def _gather_rows(n_rows, chunk):
    n_chunks = n_rows // (_NW * chunk)
    mesh = plsc.VectorSubcoreMesh(core_axis_name="c", subcore_axis_name="s")

    @functools.partial(
        pl.kernel, mesh=mesh,
        out_type=jax.ShapeDtypeStruct((n_rows, _CODE_DIM), jnp.float32),
        scratch_types=[
            pltpu.VMEM((chunk,), jnp.int32),
            pltpu.VMEM((chunk, _CODE_DIM), jnp.float32),
            pltpu.SemaphoreType.DMA,
        ],
    )
    def gather(cb_hbm, idx_hbm, out_hbm, idx_v, rows_v, sem):
        wid = lax.axis_index("s") * _NC + lax.axis_index("c")
        for j in range(n_chunks):
            base = wid * (n_chunks * chunk) + j * chunk
            pltpu.sync_copy(idx_hbm.at[pl.ds(base, chunk)], idx_v)
            pltpu.async_copy(cb_hbm.at[idx_v], rows_v, sem).wait()
            pltpu.sync_copy(rows_v, out_hbm.at[pl.ds(base, chunk)])

    return gather


def kernel(z, codebook):
    B, T, D = z.shape
    n_rows = B * T
    z_flat = z.reshape(-1, D)
    R = _ROWS_PER_BLOCK
    s_z = jnp.sum(z_flat.reshape(-1, 1, R, D) ** 2, axis=3)
    s_e = jnp.sum(codebook ** 2, axis=1)[None, :]

    grid = (n_rows // R,)
    oh, idx, cnt, sq = pl.pallas_call(
        _vq_body,
        grid=grid,
        in_specs=[
            pl.BlockSpec((R, D), lambda i: (i, 0)),
            pl.BlockSpec((1, 1, R), lambda i: (i, 0, 0)),
            pl.BlockSpec((_NUM_CODE, D), lambda i: (0, 0)),
            pl.BlockSpec((1, _NUM_CODE), lambda i: (0, 0)),
        ],
        out_specs=[
            pl.BlockSpec((R, _NUM_CODE), lambda i: (i, 0)),
            pl.BlockSpec((1, 1, R), lambda i: (i, 0, 0)),
            pl.BlockSpec((1, _NUM_CODE), lambda i: (0, 0)),
            pl.BlockSpec((1, 1), lambda i: (0, 0)),
        ],
        out_shape=[
            jax.ShapeDtypeStruct((n_rows, _NUM_CODE), jnp.float32),
            jax.ShapeDtypeStruct((n_rows // R, 1, R), jnp.int32),
            jax.ShapeDtypeStruct((1, _NUM_CODE), jnp.float32),
            jax.ShapeDtypeStruct((1, 1), jnp.float32),
        ],
    )(z_flat, s_z, codebook, s_e)

    idx = idx.reshape(n_rows, 1)
    zq_st = _gather_rows(n_rows, 128)(codebook, idx.reshape(-1))

    v = sq[0, 0] / jnp.float32(n_rows * D)
    loss = v + _BETA * v
    e_mean = cnt[0] / jnp.float32(n_rows)
    perplexity = jnp.exp(-jnp.sum(e_mean * jnp.log(e_mean + 1e-10)))
    return (loss, zq_st.reshape(B, T, D), perplexity, oh, idx)


# speedup vs baseline: 1.3881x; 1.0856x over previous
import functools

import jax
import jax.numpy as jnp
from jax import lax
from jax.experimental import pallas as pl
from jax.experimental.pallas import tpu as pltpu
from jax.experimental.pallas import tpu_sc as plsc

_NUM_CODE = 1024
_CODE_DIM = 256
_BETA = 0.1
_ROWS_PER_BLOCK = 1024

_SC_INFO = plsc.get_sparse_core_info()
_NC = _SC_INFO.num_cores
_NS = _SC_INFO.num_subcores
_NW = _NC * _NS


def _vq_body(z_ref, cb_ref,
             oh_ref, idx_ref, cnt_ref, sq_ref, se_ref):
    i = pl.program_id(0)
    z = z_ref[...]
    cb = cb_ref[...]

    @pl.when(i == 0)
    def _():
        se_col = jnp.sum(cb * cb, axis=1, keepdims=True)
        se_ref[...] = se_col.reshape(1, -1)

    mm = jax.lax.dot_general(z, cb, (((1,), (1,)), ((), ())),
                             preferred_element_type=jnp.float32)
    sz_col = jnp.sum(z * z, axis=1, keepdims=True)
    d = (sz_col + se_ref[...]) - 2.0 * mm
    m = jnp.min(d, axis=1, keepdims=True)
    lane = jax.lax.broadcasted_iota(jnp.int32, d.shape, 1)
    idx = jnp.min(jnp.where(d == m, lane, _NUM_CODE), axis=1, keepdims=True)
    oh = (lane == idx).astype(jnp.float32)
    oh_ref[...] = oh
    idx_ref[...] = idx.reshape(1, 1, -1)
    cnt = jnp.sum(oh, axis=0, keepdims=True)
    sq = jnp.sum(m).reshape(1, 1)

    @pl.when(i == 0)
    def _():
        cnt_ref[...] = cnt
        sq_ref[...] = sq

    @pl.when(i != 0)
    def _():
        cnt_ref[...] += cnt
        sq_ref[...] += sq


def _gather_rows(n_rows, chunk):
    n_chunks = n_rows // (_NW * chunk)
    mesh = plsc.VectorSubcoreMesh(core_axis_name="c", subcore_axis_name="s")

    @functools.partial(
        pl.kernel, mesh=mesh,
        out_type=jax.ShapeDtypeStruct((n_rows, _CODE_DIM), jnp.float32),
        scratch_types=[
            pltpu.VMEM((chunk,), jnp.int32),
            pltpu.VMEM((chunk, _CODE_DIM), jnp.float32),
            pltpu.SemaphoreType.DMA,
        ],
    )
    def gather(cb_hbm, idx_hbm, out_hbm, idx_v, rows_v, sem):
        wid = lax.axis_index("s") * _NC + lax.axis_index("c")
        for j in range(n_chunks):
            base = wid * (n_chunks * chunk) + j * chunk
            pltpu.sync_copy(idx_hbm.at[pl.ds(base, chunk)], idx_v)
            pltpu.async_copy(cb_hbm.at[idx_v], rows_v, sem).wait()
            pltpu.sync_copy(rows_v, out_hbm.at[pl.ds(base, chunk)])

    return gather


def kernel(z, codebook):
    B, T, D = z.shape
    n_rows = B * T
    z_flat = z.reshape(-1, D)
    R = _ROWS_PER_BLOCK
    grid = (n_rows // R,)
    oh, idx, cnt, sq = pl.pallas_call(
        _vq_body,
        grid=grid,
        in_specs=[
            pl.BlockSpec((R, D), lambda i: (i, 0)),
            pl.BlockSpec((_NUM_CODE, D), lambda i: (0, 0)),
        ],
        scratch_shapes=[pltpu.VMEM((1, _NUM_CODE), jnp.float32)],
        out_specs=[
            pl.BlockSpec((R, _NUM_CODE), lambda i: (i, 0)),
            pl.BlockSpec((1, 1, R), lambda i: (i, 0, 0)),
            pl.BlockSpec((1, _NUM_CODE), lambda i: (0, 0)),
            pl.BlockSpec((1, 1), lambda i: (0, 0)),
        ],
        out_shape=[
            jax.ShapeDtypeStruct((n_rows, _NUM_CODE), jnp.float32),
            jax.ShapeDtypeStruct((n_rows // R, 1, R), jnp.int32),
            jax.ShapeDtypeStruct((1, _NUM_CODE), jnp.float32),
            jax.ShapeDtypeStruct((1, 1), jnp.float32),
        ],
    )(z_flat, codebook)

    idx = idx.reshape(n_rows, 1)
    zq_st = _gather_rows(n_rows, 128)(codebook, idx.reshape(-1))

    v = sq[0, 0] / jnp.float32(n_rows * D)
    loss = v + _BETA * v
    e_mean = cnt[0] / jnp.float32(n_rows)
    perplexity = jnp.exp(-jnp.sum(e_mean * jnp.log(e_mean + 1e-10)))
    return (loss, zq_st.reshape(B, T, D), perplexity, oh, idx)


# speedup vs baseline: 1.3890x; 1.0007x over previous
import functools

import jax
import jax.numpy as jnp
from jax import lax
from jax.experimental import pallas as pl
from jax.experimental.pallas import tpu as pltpu
from jax.experimental.pallas import tpu_sc as plsc

_NUM_CODE = 1024
_CODE_DIM = 256
_BETA = 0.1
_ROWS_PER_BLOCK = 1024

_SC_INFO = plsc.get_sparse_core_info()
_NC = _SC_INFO.num_cores
_NS = _SC_INFO.num_subcores
_NW = _NC * _NS


def _vq_body(z_ref, cb_ref,
             oh_ref, idx_ref, cnt_ref, sq_ref, se_ref):
    i = pl.program_id(0)
    z = z_ref[...]
    cb = cb_ref[...]

    @pl.when(i == 0)
    def _():
        se_col = jnp.sum(cb * cb, axis=1, keepdims=True)
        se_ref[...] = se_col.reshape(1, -1)

    mm = jax.lax.dot_general(z, cb, (((1,), (1,)), ((), ())),
                             preferred_element_type=jnp.float32)
    sz_col = jnp.sum(z * z, axis=1, keepdims=True)
    d = (sz_col + se_ref[...]) - 2.0 * mm
    m = jnp.min(d, axis=1, keepdims=True)
    lane = jax.lax.broadcasted_iota(jnp.int32, d.shape, 1)
    idx = jnp.min(jnp.where(d == m, lane, _NUM_CODE), axis=1, keepdims=True)
    oh = (lane == idx).astype(jnp.float32)
    oh_ref[...] = oh
    idx_ref[...] = idx.reshape(1, 1, -1)
    cnt = jnp.sum(oh, axis=0, keepdims=True)
    sq = jnp.sum(m).reshape(1, 1)

    @pl.when(i == 0)
    def _():
        cnt_ref[...] = cnt
        sq_ref[...] = sq

    @pl.when(i != 0)
    def _():
        cnt_ref[...] += cnt
        sq_ref[...] += sq


def _gather_rows(n_rows, chunk):
    n_chunks = n_rows // (_NW * chunk)
    mesh = plsc.VectorSubcoreMesh(core_axis_name="c", subcore_axis_name="s")

    @functools.partial(
        pl.kernel, mesh=mesh,
        out_type=jax.ShapeDtypeStruct((n_rows, _CODE_DIM), jnp.float32),
        scratch_types=[
            pltpu.VMEM((n_chunks, chunk), jnp.int32),
            pltpu.VMEM((n_chunks, chunk, _CODE_DIM), jnp.float32),
            pltpu.SemaphoreType.DMA,
            pltpu.SemaphoreType.DMA,
        ],
    )
    def gather(cb_hbm, idx_hbm, out_hbm, idx_v, rows_v, gsem, ssem):
        wid = lax.axis_index("s") * _NC + lax.axis_index("c")
        base = wid * (n_chunks * chunk)
        for j in range(n_chunks):
            pltpu.sync_copy(idx_hbm.at[pl.ds(base + j * chunk, chunk)],
                            idx_v.at[j])
        gathers = [pltpu.async_copy(cb_hbm.at[idx_v.at[j]], rows_v.at[j], gsem)
                   for j in range(n_chunks)]
        stores = []
        for j in range(n_chunks):
            gathers[j].wait()
            stores.append(pltpu.async_copy(
                rows_v.at[j], out_hbm.at[pl.ds(base + j * chunk, chunk)], ssem))
        for s in stores:
            s.wait()

    return gather


def kernel(z, codebook):
    B, T, D = z.shape
    n_rows = B * T
    z_flat = z.reshape(-1, D)
    R = _ROWS_PER_BLOCK
    grid = (n_rows // R,)
    oh, idx, cnt, sq = pl.pallas_call(
        _vq_body,
        grid=grid,
        in_specs=[
            pl.BlockSpec((R, D), lambda i: (i, 0)),
            pl.BlockSpec((_NUM_CODE, D), lambda i: (0, 0)),
        ],
        scratch_shapes=[pltpu.VMEM((1, _NUM_CODE), jnp.float32)],
        out_specs=[
            pl.BlockSpec((R, _NUM_CODE), lambda i: (i, 0)),
            pl.BlockSpec((1, 1, R), lambda i: (i, 0, 0)),
            pl.BlockSpec((1, _NUM_CODE), lambda i: (0, 0)),
            pl.BlockSpec((1, 1), lambda i: (0, 0)),
        ],
        out_shape=[
            jax.ShapeDtypeStruct((n_rows, _NUM_CODE), jnp.float32),
            jax.ShapeDtypeStruct((n_rows // R, 1, R), jnp.int32),
            jax.ShapeDtypeStruct((1, _NUM_CODE), jnp.float32),
            jax.ShapeDtypeStruct((1, 1), jnp.float32),
        ],
    )(z_flat, codebook)

    idx = idx.reshape(n_rows, 1)
    zq_st = _gather_rows(n_rows, 128)(codebook, idx.reshape(-1))

    v = sq[0, 0] / jnp.float32(n_rows * D)
    loss = v + _BETA * v
    e_mean = cnt[0] / jnp.float32(n_rows)
    perplexity = jnp.exp(-jnp.sum(e_mean * jnp.log(e_mean + 1e-10)))
    return (loss, zq_st.reshape(B, T, D), perplexity, oh, idx)
